# baseline (speedup 1.0000x reference)
import functools

import jax
import jax.numpy as jnp
from jax import lax
from jax.experimental import pallas as pl
from jax.experimental.pallas import tpu as pltpu
from jax.experimental.pallas import tpu_sc as plsc

NUM_TOKENS = 8192
DIM = 768

_info = plsc.get_sparse_core_info()
_NC = _info.num_cores
_NS = _info.num_subcores
_NW = _NC * _NS
_ROWS_PER_W = NUM_TOKENS // _NW
_RA = 128
_RB = 32
_SC_ROWS_PER_W = _RA + _RB
_TC_ROWS_PER_W = _ROWS_PER_W - _SC_ROWS_PER_W


@functools.partial(
    pl.kernel,
    out_type=jax.ShapeDtypeStruct((NUM_TOKENS, 1, DIM), jnp.float32),
    mesh=plsc.VectorSubcoreMesh(core_axis_name="c", subcore_axis_name="s"),
    scratch_types=(
        [pltpu.VMEM((_RA, DIM), jnp.float32),
         pltpu.VMEM_SHARED((_NS, _RB, DIM), jnp.float32)]
        + [pltpu.SemaphoreType.DMA] * 4
    ),
)
def _identity_rows_sc(table_hbm, out_hbm, buf_a, buf_b,
                      sa_in, sb_in, sa_out, sb_out):
    sid = lax.axis_index("s")
    wid = sid * _NC + lax.axis_index("c")
    base = wid * _ROWS_PER_W
    base_b = base + _RA

    in_a = pltpu.async_copy(table_hbm.at[pl.ds(base, _RA)], buf_a, sa_in)
    in_b = pltpu.async_copy(
        table_hbm.at[pl.ds(base_b, _RB)], buf_b.at[sid], sb_in)
    in_a.wait()
    out_a = pltpu.async_copy(
        buf_a, out_hbm.at[pl.ds(base, _RA), 0], sa_out)
    in_b.wait()
    out_b = pltpu.async_copy(
        buf_b.at[sid], out_hbm.at[pl.ds(base_b, _RB), 0], sb_out)
    out_a.wait()
    out_b.wait()


_TC_BS = 32
_TC_J = _TC_ROWS_PER_W // _TC_BS


def _tc_index(i, j):
    return (_ROWS_PER_W // _TC_BS) * i + _SC_ROWS_PER_W // _TC_BS + j


def _tc_copy(t_ref, _alias_ref, o_ref):
    o_ref[...] = t_ref[...][:, None, :]


def kernel(table):
    partial = _identity_rows_sc(table)
    return pl.pallas_call(
        _tc_copy,
        grid=(_NW, _TC_J),
        in_specs=[
            pl.BlockSpec((_TC_BS, DIM), lambda i, j: (_tc_index(i, j), 0)),
            pl.BlockSpec((_TC_BS, 1, DIM),
                         lambda i, j: (_tc_index(i, j), 0, 0)),
        ],
        out_specs=pl.BlockSpec((_TC_BS, 1, DIM),
                               lambda i, j: (_tc_index(i, j), 0, 0)),
        out_shape=jax.ShapeDtypeStruct((NUM_TOKENS, 1, DIM), jnp.float32),
        input_output_aliases={1: 0},
    )(table, partial)


# speedup vs baseline: 2.0047x; 2.0047x over previous
import functools

import jax
import jax.numpy as jnp
from jax import lax
from jax.experimental import pallas as pl
from jax.experimental.pallas import tpu as pltpu
from jax.experimental.pallas import tpu_sc as plsc

NUM_TOKENS = 8192
DIM = 768

_info = plsc.get_sparse_core_info()
_NC = _info.num_cores
_NS = _info.num_subcores
_NW = _NC * _NS
_RA = 128
_RB = 32
_ROWS_PER_W = _RA + _RB
_SC_ROWS = _NW * _ROWS_PER_W
_TC_BS = 256
_TC_OFF = _SC_ROWS // _TC_BS
_TC_BLOCKS = (NUM_TOKENS - _SC_ROWS) // _TC_BS


@functools.partial(
    pl.kernel,
    out_type=jax.ShapeDtypeStruct((NUM_TOKENS, 1, DIM), jnp.float32),
    mesh=plsc.VectorSubcoreMesh(core_axis_name="c", subcore_axis_name="s"),
    scratch_types=(
        [pltpu.VMEM((_RA, DIM), jnp.float32),
         pltpu.VMEM_SHARED((_NS, _RB, DIM), jnp.float32)]
        + [pltpu.SemaphoreType.DMA] * 4
    ),
)
def _identity_rows_sc(table_hbm, out_hbm, buf_a, buf_b,
                      sa_in, sb_in, sa_out, sb_out):
    sid = lax.axis_index("s")
    wid = sid * _NC + lax.axis_index("c")
    base = wid * _ROWS_PER_W
    base_b = base + _RA

    in_a = pltpu.async_copy(table_hbm.at[pl.ds(base, _RA)], buf_a, sa_in)
    in_b = pltpu.async_copy(
        table_hbm.at[pl.ds(base_b, _RB)], buf_b.at[sid], sb_in)
    in_a.wait()
    out_a = pltpu.async_copy(
        buf_a, out_hbm.at[pl.ds(base, _RA), 0], sa_out)
    in_b.wait()
    out_b = pltpu.async_copy(
        buf_b.at[sid], out_hbm.at[pl.ds(base_b, _RB), 0], sb_out)
    out_a.wait()
    out_b.wait()


def _tc_copy(t_ref, _alias_ref, o_ref):
    o_ref[...] = t_ref[...][:, None, :]


def kernel(table):
    partial = _identity_rows_sc(table)
    return pl.pallas_call(
        _tc_copy,
        grid=(_TC_BLOCKS,),
        in_specs=[
            pl.BlockSpec((_TC_BS, DIM), lambda i: (_TC_OFF + i, 0)),
            pl.BlockSpec(memory_space=pl.ANY),
        ],
        out_specs=pl.BlockSpec((_TC_BS, 1, DIM),
                               lambda i: (_TC_OFF + i, 0, 0)),
        out_shape=jax.ShapeDtypeStruct((NUM_TOKENS, 1, DIM), jnp.float32),
        input_output_aliases={1: 0},
    )(table, partial)


# speedup vs baseline: 2.3015x; 1.1480x over previous
import functools

import jax
import jax.numpy as jnp
from jax import lax
from jax.experimental import pallas as pl
from jax.experimental.pallas import tpu as pltpu
from jax.experimental.pallas import tpu_sc as plsc

NUM_TOKENS = 8192
DIM = 768

_info = plsc.get_sparse_core_info()
_NC = _info.num_cores
_NS = _info.num_subcores
_NW = _NC * _NS
_ROWS_PER_W = NUM_TOKENS // _NW
_RA = 128
_RB = 32
_RC = _ROWS_PER_W - _RA - _RB


@functools.partial(
    pl.kernel,
    out_type=jax.ShapeDtypeStruct((NUM_TOKENS, 1, DIM), jnp.float32),
    mesh=plsc.VectorSubcoreMesh(core_axis_name="c", subcore_axis_name="s"),
    scratch_types=(
        [pltpu.VMEM((_RA, DIM), jnp.float32),
         pltpu.VMEM_SHARED((_NS, _RB, DIM), jnp.float32)]
        + [pltpu.SemaphoreType.DMA] * 6
    ),
)
def _identity_rows_sc(table_hbm, out_hbm, buf_a, buf_b,
                      sa_in, sb_in, sc_in, sa_out, sb_out, sc_out):
    sid = lax.axis_index("s")
    wid = sid * _NC + lax.axis_index("c")
    base = wid * _ROWS_PER_W
    base_b = base + _RA
    base_c = base_b + _RB

    in_a = pltpu.async_copy(table_hbm.at[pl.ds(base, _RA)], buf_a, sa_in)
    in_b = pltpu.async_copy(
        table_hbm.at[pl.ds(base_b, _RB)], buf_b.at[sid], sb_in)
    in_a.wait()
    out_a = pltpu.async_copy(
        buf_a, out_hbm.at[pl.ds(base, _RA), 0], sa_out)
    in_b.wait()
    out_b = pltpu.async_copy(
        buf_b.at[sid], out_hbm.at[pl.ds(base_b, _RB), 0], sb_out)
    out_a.wait()
    in_c = pltpu.async_copy(
        table_hbm.at[pl.ds(base_c, _RC)], buf_a.at[pl.ds(0, _RC)], sc_in)
    in_c.wait()
    out_c = pltpu.async_copy(
        buf_a.at[pl.ds(0, _RC)], out_hbm.at[pl.ds(base_c, _RC), 0], sc_out)
    out_b.wait()
    out_c.wait()


def kernel(table):
    return _identity_rows_sc(table)
